# baseline (speedup 1.0000x reference)
import jax
import jax.numpy as jnp
from jax import lax
from jax.experimental import pallas as pl
from jax.experimental.pallas import tpu as pltpu
from jax.experimental.pallas import tpu_sc as plsc

NUM_USERS = 25000
NUM_ITEMS = 25000
N_NODES = NUM_USERS + NUM_ITEMS
N_EDGES = 800000
N_LAYERS = 3

NC = 2
NS = 16
LANES = 16
NGROUPS = 4
GW = 32

CHUNK = 512
CHUNKS_PER_TILE = 98
EPAD = NS * CHUNKS_PER_TILE * CHUNK
EROWS = EPAD // 128
ROWS_PER_TILE = EROWS // NS
NPAD = 50176
NODES_PER_TILE = NPAD // NS


def _mm_body(a_ref, w_ref, o_ref):
    o_ref[...] = jnp.dot(a_ref[...], w_ref[...],
                         preferred_element_type=jnp.float32)


def _attr_matmul(a, wt):
    return pl.pallas_call(
        _mm_body,
        out_shape=jax.ShapeDtypeStruct((a.shape[0], wt.shape[1]), jnp.float32),
    )(a, wt)


def _prop_body(emb_in, src4_hbm, dst_hbm, w_hbm, emb_out, emb_scr,
               acc, src_v, dst_v, w_v, rows_v):
    c = lax.axis_index("c")
    s = lax.axis_index("s")
    lanes = lax.broadcasted_iota(jnp.int32, (LANES,), 0)

    def run_layer(g, src_tab, dst_tab):
        def chunk_body(ci, carry):
            r0 = s * ROWS_PER_TILE + ci * (CHUNK // 128)
            pltpu.sync_copy(src4_hbm.at[g, pl.ds(r0, CHUNK // 128)], src_v)
            pltpu.sync_copy(dst_hbm.at[pl.ds(r0, CHUNK // 128)], dst_v)
            pltpu.sync_copy(w_hbm.at[pl.ds(r0 * 128, CHUNK)], w_v)
            for j in range(CHUNK // 128):
                pltpu.sync_copy(src_tab.at[src_v.at[j]],
                                rows_v.at[pl.ds(j * 128, 128)])

            def scale_body(k, carry2):
                e0 = k * LANES
                wv = w_v[pl.ds(e0, LANES)]
                eidx = e0 + lanes
                for col in range(GW):
                    cidx = jnp.full((LANES,), col, jnp.int32)
                    vals = plsc.load_gather(rows_v, [eidx, cidx])
                    plsc.store_scatter(rows_v, [eidx, cidx], vals * wv)
                return carry2

            lax.fori_loop(0, CHUNK // LANES, scale_body, 0)

            for j in range(CHUNK // 128):
                pltpu.sync_copy(rows_v.at[pl.ds(j * 128, 128)],
                                acc.at[dst_v.at[j]], add=True)
            return carry

        lax.fori_loop(0, CHUNKS_PER_TILE, chunk_body, 0)
        plsc.subcore_barrier()
        pltpu.sync_copy(
            acc.at[pl.ds(s * NODES_PER_TILE, NODES_PER_TILE)],
            dst_tab.at[pl.ds(g * NPAD + s * NODES_PER_TILE,
                             NODES_PER_TILE)])
        plsc.subcore_barrier()

    for p in range(NGROUPS // NC):
        g = c * (NGROUPS // NC) + p
        pltpu.sync_copy(
            emb_in.at[pl.ds(g * NPAD + s * NODES_PER_TILE,
                            NODES_PER_TILE)],
            acc.at[pl.ds(s * NODES_PER_TILE, NODES_PER_TILE)])
        plsc.subcore_barrier()
        run_layer(g, emb_in, emb_out)
        run_layer(g, emb_out, emb_scr)
        run_layer(g, emb_scr, emb_out)


@jax.jit
def _propagate(emb4, src4, dst2d, w1d):
    mesh = plsc.VectorSubcoreMesh(core_axis_name="c", subcore_axis_name="s")
    f = pl.kernel(
        _prop_body,
        out_type=(
            jax.ShapeDtypeStruct((NGROUPS * NPAD, GW), jnp.float32),
            jax.ShapeDtypeStruct((NGROUPS * NPAD, GW), jnp.float32),
        ),
        mesh=mesh,
        compiler_params=pltpu.CompilerParams(
            needs_layout_passes=False, use_tc_tiling_on_sc=False),
        scratch_types=[
            pltpu.VMEM_SHARED((NPAD, GW), jnp.float32),
            pltpu.VMEM((CHUNK // 128, 128), jnp.int32),
            pltpu.VMEM((CHUNK // 128, 128), jnp.int32),
            pltpu.VMEM((CHUNK,), jnp.float32),
            pltpu.VMEM((CHUNK, GW), jnp.float32),
        ],
    )
    return f(emb4, src4, dst2d, w1d)


def kernel(missing_attr, user_emb, item_emb, trans_w, edge_weight, edge_index):
    attr = _attr_matmul(missing_attr, trans_w.T)
    emb = jnp.concatenate(
        [user_emb, jnp.concatenate([item_emb, attr], axis=1)], axis=0)
    emb = jnp.pad(emb, ((0, NPAD - N_NODES), (0, 0)))
    emb4 = emb.reshape(NPAD, NGROUPS, GW).transpose(1, 0, 2)
    emb4 = emb4.reshape(NGROUPS * NPAD, GW)

    pad = EPAD - N_EDGES
    src = jnp.concatenate([edge_index[0], jnp.zeros((pad,), jnp.int32)])
    dst = jnp.concatenate([edge_index[1], jnp.zeros((pad,), jnp.int32)])
    w = jnp.concatenate([edge_weight, jnp.zeros((pad,), jnp.float32)])
    goff = (jnp.arange(NGROUPS, dtype=jnp.int32) * NPAD)[:, None]
    src4 = (src[None, :] + goff).reshape(NGROUPS, EROWS, 128)
    dst2d = dst.reshape(EROWS, 128)

    out, _ = _propagate(emb4, src4, dst2d, w)
    final = out.reshape(NGROUPS, NPAD, GW).transpose(1, 0, 2)
    final = final.reshape(NPAD, NGROUPS * GW)
    return final[:NUM_USERS], final[NUM_USERS:N_NODES]


# speedup vs baseline: 1.1193x; 1.1193x over previous
import jax
import jax.numpy as jnp
from jax import lax
from jax.experimental import pallas as pl
from jax.experimental.pallas import tpu as pltpu
from jax.experimental.pallas import tpu_sc as plsc

NUM_USERS = 25000
NUM_ITEMS = 25000
N_NODES = NUM_USERS + NUM_ITEMS
N_EDGES = 800000
N_LAYERS = 3

NC = 2
NS = 16
LANES = 16
NGROUPS = 4
GW = 32

CHUNK = 512
CHUNKS_PER_TILE = 98
EPAD = NS * CHUNKS_PER_TILE * CHUNK
EROWS = EPAD // 128
ROWS_PER_TILE = EROWS // NS
NPAD = 50176
NODES_PER_TILE = NPAD // NS


def _mm_body(a_ref, w_ref, o_ref):
    o_ref[...] = jnp.dot(a_ref[...], w_ref[...],
                         preferred_element_type=jnp.float32)


def _attr_matmul(a, wt):
    return pl.pallas_call(
        _mm_body,
        out_shape=jax.ShapeDtypeStruct((a.shape[0], wt.shape[1]), jnp.float32),
    )(a, wt)


def _prop_body(emb_in, src4_hbm, dst_hbm, w_hbm, emb_out, emb_scr,
               acc, src_v, dst_v, w_v, rows_v, sem_l, sem_s):
    c = lax.axis_index("c")
    s = lax.axis_index("s")
    lanes = lax.broadcasted_iota(jnp.int32, (LANES,), 0)

    def run_layer(g, src_tab, dst_tab):
        def chunk_body(ci, carry):
            r0 = s * ROWS_PER_TILE + ci * (CHUNK // 128)
            d1 = pltpu.async_copy(src4_hbm.at[g, pl.ds(r0, CHUNK // 128)],
                                  src_v, sem_l)
            d2 = pltpu.async_copy(dst_hbm.at[pl.ds(r0, CHUNK // 128)],
                                  dst_v, sem_l)
            d3 = pltpu.async_copy(w_hbm.at[pl.ds(r0 * 128, CHUNK)], w_v, sem_l)
            d1.wait(); d2.wait(); d3.wait()
            gds = [pltpu.async_copy(src_tab.at[src_v.at[j]],
                                    rows_v.at[pl.ds(j * 128, 128)], sem_l)
                   for j in range(CHUNK // 128)]
            for d in gds:
                d.wait()

            def scale_body(k, carry2):
                e0 = k * LANES
                wv = w_v[pl.ds(e0, LANES)]
                eidx = e0 + lanes
                for col in range(GW):
                    cidx = jnp.full((LANES,), col, jnp.int32)
                    vals = plsc.load_gather(rows_v, [eidx, cidx])
                    plsc.store_scatter(rows_v, [eidx, cidx], vals * wv)
                return carry2

            lax.fori_loop(0, CHUNK // LANES, scale_body, 0)

            sds = [pltpu.async_copy(rows_v.at[pl.ds(j * 128, 128)],
                                    acc.at[dst_v.at[j]], sem_s, add=True)
                   for j in range(CHUNK // 128)]
            for d in sds:
                d.wait()
            return carry

        lax.fori_loop(0, CHUNKS_PER_TILE, chunk_body, 0)
        plsc.subcore_barrier()
        pltpu.sync_copy(
            acc.at[pl.ds(s * NODES_PER_TILE, NODES_PER_TILE)],
            dst_tab.at[pl.ds(g * NPAD + s * NODES_PER_TILE,
                             NODES_PER_TILE)])
        plsc.subcore_barrier()

    for p in range(NGROUPS // NC):
        g = c * (NGROUPS // NC) + p
        pltpu.sync_copy(
            emb_in.at[pl.ds(g * NPAD + s * NODES_PER_TILE,
                            NODES_PER_TILE)],
            acc.at[pl.ds(s * NODES_PER_TILE, NODES_PER_TILE)])
        plsc.subcore_barrier()
        run_layer(g, emb_in, emb_out)
        run_layer(g, emb_out, emb_scr)
        run_layer(g, emb_scr, emb_out)


@jax.jit
def _propagate(emb4, src4, dst2d, w1d):
    mesh = plsc.VectorSubcoreMesh(core_axis_name="c", subcore_axis_name="s")
    f = pl.kernel(
        _prop_body,
        out_type=(
            jax.ShapeDtypeStruct((NGROUPS * NPAD, GW), jnp.float32),
            jax.ShapeDtypeStruct((NGROUPS * NPAD, GW), jnp.float32),
        ),
        mesh=mesh,
        compiler_params=pltpu.CompilerParams(
            needs_layout_passes=False, use_tc_tiling_on_sc=False),
        scratch_types=[
            pltpu.VMEM_SHARED((NPAD, GW), jnp.float32),
            pltpu.VMEM((CHUNK // 128, 128), jnp.int32),
            pltpu.VMEM((CHUNK // 128, 128), jnp.int32),
            pltpu.VMEM((CHUNK,), jnp.float32),
            pltpu.VMEM((CHUNK, GW), jnp.float32),
            pltpu.SemaphoreType.DMA,
            pltpu.SemaphoreType.DMA,
        ],
    )
    return f(emb4, src4, dst2d, w1d)


def kernel(missing_attr, user_emb, item_emb, trans_w, edge_weight, edge_index):
    attr = _attr_matmul(missing_attr, trans_w.T)
    emb = jnp.concatenate(
        [user_emb, jnp.concatenate([item_emb, attr], axis=1)], axis=0)
    emb = jnp.pad(emb, ((0, NPAD - N_NODES), (0, 0)))
    emb4 = emb.reshape(NPAD, NGROUPS, GW).transpose(1, 0, 2)
    emb4 = emb4.reshape(NGROUPS * NPAD, GW)

    pad = EPAD - N_EDGES
    src = jnp.concatenate([edge_index[0], jnp.zeros((pad,), jnp.int32)])
    dst = jnp.concatenate([edge_index[1], jnp.zeros((pad,), jnp.int32)])
    w = jnp.concatenate([edge_weight, jnp.zeros((pad,), jnp.float32)])
    goff = (jnp.arange(NGROUPS, dtype=jnp.int32) * NPAD)[:, None]
    src4 = (src[None, :] + goff).reshape(NGROUPS, EROWS, 128)
    dst2d = dst.reshape(EROWS, 128)

    out, _ = _propagate(emb4, src4, dst2d, w)
    final = out.reshape(NGROUPS, NPAD, GW).transpose(1, 0, 2)
    final = final.reshape(NPAD, NGROUPS * GW)
    return final[:NUM_USERS], final[NUM_USERS:N_NODES]
